# initial kernel scaffold (unmeasured)
import jax
import jax.numpy as jnp
from jax import lax
from jax.experimental import pallas as pl
from jax.experimental.pallas import tpu as pltpu


def kernel(
    x,
):
    def body(*refs):
        pass

    out_shape = jax.ShapeDtypeStruct(..., jnp.float32)
    return pl.pallas_call(body, out_shape=out_shape)(...)


# baseline (device time: 700765 ns/iter reference)
import jax
import jax.numpy as jnp
from jax import lax
from jax.experimental import pallas as pl
from jax.experimental.pallas import tpu as pltpu

N_DEV = 8


def kernel(x):
    m_per, n = x.shape

    def body(x_ref, out_ref, stage_ref, local_sem, send_sems, recv_sems):
        my = lax.axis_index("i")
        left = (my - 1) % N_DEV
        right = (my + 1) % N_DEV

        barrier_sem = pltpu.get_barrier_semaphore()
        for nbr in (left, right):
            pl.semaphore_signal(
                barrier_sem, inc=1,
                device_id=(nbr,), device_id_type=pl.DeviceIdType.MESH,
            )
        pl.semaphore_wait(barrier_sem, 2)

        stage_ref[...] = x_ref[...].astype(jnp.bfloat16)
        cp = pltpu.make_async_copy(
            stage_ref, out_ref.at[pl.ds(my * m_per, m_per)], local_sem
        )
        cp.start()
        cp.wait()

        for h in range(N_DEV - 1):
            src_origin = (my - h) % N_DEV
            rdma = pltpu.make_async_remote_copy(
                src_ref=out_ref.at[pl.ds(src_origin * m_per, m_per)],
                dst_ref=out_ref.at[pl.ds(src_origin * m_per, m_per)],
                send_sem=send_sems.at[h],
                recv_sem=recv_sems.at[h],
                device_id=(right,),
                device_id_type=pl.DeviceIdType.MESH,
            )
            rdma.start()
            rdma.wait()

    return pl.pallas_call(
        body,
        out_shape=jax.ShapeDtypeStruct((N_DEV * m_per, n), jnp.bfloat16),
        in_specs=[pl.BlockSpec(memory_space=pltpu.VMEM)],
        out_specs=pl.BlockSpec(memory_space=pl.ANY),
        scratch_shapes=[
            pltpu.VMEM((m_per, n), jnp.bfloat16),
            pltpu.SemaphoreType.DMA,
            pltpu.SemaphoreType.DMA((N_DEV - 1,)),
            pltpu.SemaphoreType.DMA((N_DEV - 1,)),
        ],
        compiler_params=pltpu.CompilerParams(collective_id=0),
    )(x)


# device time: 387913 ns/iter; 1.8065x vs baseline; 1.8065x over previous
import jax
import jax.numpy as jnp
from jax import lax
from jax.experimental import pallas as pl
from jax.experimental.pallas import tpu as pltpu

N_DEV = 8


def kernel(x):
    m_per, n = x.shape
    m_half = m_per // 2

    def body(x_ref, out_ref, stage_ref, local_sem,
             send_cw, recv_cw, send_ccw, recv_ccw):
        my = lax.axis_index("i")
        left = (my - 1) % N_DEV
        right = (my + 1) % N_DEV

        barrier_sem = pltpu.get_barrier_semaphore()
        for nbr in (left, right):
            pl.semaphore_signal(
                barrier_sem, inc=1,
                device_id=(nbr,), device_id_type=pl.DeviceIdType.MESH,
            )
        pl.semaphore_wait(barrier_sem, 2)

        stage_ref[...] = x_ref[...].astype(jnp.bfloat16)
        cp = pltpu.make_async_copy(
            stage_ref, out_ref.at[pl.ds(my * m_per, m_per)], local_sem
        )
        cp.start()
        cp.wait()

        pending = []
        for h in range(N_DEV - 1):
            o_cw = (my - h) % N_DEV
            o_ccw = (my + h) % N_DEV
            cw = pltpu.make_async_remote_copy(
                src_ref=out_ref.at[pl.ds(o_cw * m_per, m_half)],
                dst_ref=out_ref.at[pl.ds(o_cw * m_per, m_half)],
                send_sem=send_cw.at[h],
                recv_sem=recv_cw.at[h],
                device_id=(right,),
                device_id_type=pl.DeviceIdType.MESH,
            )
            ccw = pltpu.make_async_remote_copy(
                src_ref=out_ref.at[pl.ds(o_ccw * m_per + m_half, m_half)],
                dst_ref=out_ref.at[pl.ds(o_ccw * m_per + m_half, m_half)],
                send_sem=send_ccw.at[h],
                recv_sem=recv_ccw.at[h],
                device_id=(left,),
                device_id_type=pl.DeviceIdType.MESH,
            )
            cw.start()
            ccw.start()
            cw.wait_recv()
            ccw.wait_recv()
            pending.append(cw)
            pending.append(ccw)
        for d in pending:
            d.wait_send()

    return pl.pallas_call(
        body,
        out_shape=jax.ShapeDtypeStruct((N_DEV * m_per, n), jnp.bfloat16),
        in_specs=[pl.BlockSpec(memory_space=pltpu.VMEM)],
        out_specs=pl.BlockSpec(memory_space=pl.ANY),
        scratch_shapes=[
            pltpu.VMEM((m_per, n), jnp.bfloat16),
            pltpu.SemaphoreType.DMA,
            pltpu.SemaphoreType.DMA((N_DEV - 1,)),
            pltpu.SemaphoreType.DMA((N_DEV - 1,)),
            pltpu.SemaphoreType.DMA((N_DEV - 1,)),
            pltpu.SemaphoreType.DMA((N_DEV - 1,)),
        ],
        compiler_params=pltpu.CompilerParams(collective_id=0),
    )(x)


# device time: 370825 ns/iter; 1.8897x vs baseline; 1.0461x over previous
import jax
import jax.numpy as jnp
from jax import lax
from jax.experimental import pallas as pl
from jax.experimental.pallas import tpu as pltpu

N_DEV = 8
SUB = 2


def kernel(x):
    m_per, n = x.shape
    m_half = m_per // 2
    m_sub = m_half // SUB

    def body(x_ref, out_ref, stage_ref, local_sem,
             send_cw, recv_cw, send_ccw, recv_ccw):
        my = lax.axis_index("i")
        left = (my - 1) % N_DEV
        right = (my + 1) % N_DEV

        def row_cw(o, j):
            return o * m_per + j * m_sub

        def row_ccw(o, j):
            return o * m_per + m_half + j * m_sub

        barrier_sem = pltpu.get_barrier_semaphore()
        for nbr in (left, right):
            pl.semaphore_signal(
                barrier_sem, inc=1,
                device_id=(nbr,), device_id_type=pl.DeviceIdType.MESH,
            )
        pl.semaphore_wait(barrier_sem, 2)

        stage_ref[...] = x_ref[...].astype(jnp.bfloat16)
        cp = pltpu.make_async_copy(
            stage_ref, out_ref.at[pl.ds(my * m_per, m_per)], local_sem
        )
        cp.start()

        def send_desc(h, j, direction):
            if direction == "cw":
                o = (my - h) % N_DEV
                dst = out_ref.at[pl.ds(row_cw(o, j), m_sub)]
                src = (stage_ref.at[pl.ds(j * m_sub, m_sub)] if h == 0
                       else out_ref.at[pl.ds(row_cw(o, j), m_sub)])
                sem, dev = send_cw, right
            else:
                o = (my + h) % N_DEV
                dst = out_ref.at[pl.ds(row_ccw(o, j), m_sub)]
                src = (stage_ref.at[pl.ds(m_half + j * m_sub, m_sub)] if h == 0
                       else out_ref.at[pl.ds(row_ccw(o, j), m_sub)])
                sem, dev = send_ccw, left
            return pltpu.make_async_remote_copy(
                src_ref=src, dst_ref=dst,
                send_sem=sem.at[h, j], recv_sem=(recv_cw if direction == "cw"
                                                else recv_ccw).at[h, j],
                device_id=(dev,), device_id_type=pl.DeviceIdType.MESH,
            )

        def recv_desc(h, j, direction):
            if direction == "cw":
                o = (my - h - 1) % N_DEV
                dst = out_ref.at[pl.ds(row_cw(o, j), m_sub)]
                sem = recv_cw
            else:
                o = (my + h + 1) % N_DEV
                dst = out_ref.at[pl.ds(row_ccw(o, j), m_sub)]
                sem = recv_ccw
            return pltpu.make_async_remote_copy(
                src_ref=dst, dst_ref=dst,
                send_sem=(send_cw if direction == "cw" else send_ccw).at[h, j],
                recv_sem=sem.at[h, j],
                device_id=(left if direction == "cw" else right,),
                device_id_type=pl.DeviceIdType.MESH,
            )

        pending = []
        for j in range(SUB):
            for d in ("cw", "ccw"):
                s = send_desc(0, j, d)
                s.start()
                pending.append(s)
        for h in range(N_DEV - 1):
            for j in range(SUB):
                for d in ("cw", "ccw"):
                    recv_desc(h, j, d).wait_recv()
                    if h < N_DEV - 2:
                        s = send_desc(h + 1, j, d)
                        s.start()
                        pending.append(s)
        for s in pending:
            s.wait_send()
        cp.wait()

    return pl.pallas_call(
        body,
        out_shape=jax.ShapeDtypeStruct((N_DEV * m_per, n), jnp.bfloat16),
        in_specs=[pl.BlockSpec(memory_space=pltpu.VMEM)],
        out_specs=pl.BlockSpec(memory_space=pl.ANY),
        scratch_shapes=[
            pltpu.VMEM((m_per, n), jnp.bfloat16),
            pltpu.SemaphoreType.DMA,
            pltpu.SemaphoreType.DMA((N_DEV - 1, SUB)),
            pltpu.SemaphoreType.DMA((N_DEV - 1, SUB)),
            pltpu.SemaphoreType.DMA((N_DEV - 1, SUB)),
            pltpu.SemaphoreType.DMA((N_DEV - 1, SUB)),
        ],
        compiler_params=pltpu.CompilerParams(collective_id=0),
    )(x)
